# baseline (speedup 1.0000x reference)
import functools

import jax
import jax.numpy as jnp
from jax import lax
from jax.experimental import pallas as pl
from jax.experimental.pallas import tpu as pltpu
from jax.experimental.pallas import tpu_sc as plsc

VOCAB = 100000
B = 4096
L = 200

_INFO = plsc.get_sparse_core_info()
_NC = _INFO.num_cores
_NS = _INFO.num_subcores
_NW = _NC * _NS
_LANES = _INFO.num_lanes

_ROWS_PER_W = B // _NW
_CHUNK_ROWS = 32
_NCHUNK = _ROWS_PER_W // _CHUNK_ROWS
_GPC = _CHUNK_ROWS // _LANES
_FULL = L // _LANES
_TAIL = L - _FULL * _LANES
_TSTRIDE = _LANES + 1


def _sc_body(ids_hbm, table_hbm, out_hbm, table_sp, table_v, ids_a, ids_b,
             tbuf, out_v, sem_t, sem_a, sem_b):
    sid = lax.axis_index("s")
    wid = sid * _NC + lax.axis_index("c")

    @pl.when(sid == 0)
    def _():
        pltpu.sync_copy(table_hbm, table_sp)

    plsc.subcore_barrier()
    tcopy = pltpu.make_async_copy(table_sp, table_v, sem_t)
    tcopy.start()

    base_row = pl.multiple_of(wid * _ROWS_PER_W, 8)
    bufs = [ids_a, ids_b]
    sems = [sem_a, sem_b]

    def chunk_copy(c):
        return pltpu.make_async_copy(
            ids_hbm.at[pl.ds(base_row + c * _CHUNK_ROWS, _CHUNK_ROWS), :],
            bufs[c % 2], sems[c % 2])

    chunk_copy(0).start()
    chunk_copy(1).start()

    lane = lax.broadcasted_iota(jnp.int32, (_LANES,), 0)
    tail_mask = lane >= (_LANES - _TAIL)
    zero = jnp.zeros((_LANES,), jnp.float32)

    for c in range(_NCHUNK):
        chunk_copy(c).wait()
        if c == 0:
            tcopy.wait()
        buf = bufs[c % 2]

        for g in range(_GPC):
            def row_body(a, carry, buf=buf, g=g):
                r = g * _LANES + a
                acc = zero
                for cc in range(_FULL):
                    ids16 = buf[r, pl.ds(cc * _LANES, _LANES)]
                    acc = acc + plsc.load_gather(table_v, [ids16])
                ids_t = buf[r, pl.ds(L - _LANES, _LANES)]
                vals_t = plsc.load_gather(table_v, [ids_t])
                acc = acc + jnp.where(tail_mask, vals_t, zero)
                tbuf[pl.ds(a * _TSTRIDE, _LANES)] = acc
                return carry

            lax.fori_loop(0, _LANES, row_body, 0, unroll=2)

            tot = zero
            for k in range(_LANES):
                tot = tot + plsc.load_gather(tbuf, [lane * _TSTRIDE + k])
            out_v[pl.ds(c * _CHUNK_ROWS + g * _LANES, _LANES)] = tot

        if c + 2 < _NCHUNK:
            chunk_copy(c + 2).start()

    base_out = pl.multiple_of(wid * _ROWS_PER_W, 8)
    pltpu.sync_copy(out_v, out_hbm.at[pl.ds(base_out, _ROWS_PER_W)])


@jax.jit
def kernel(predicted_ids, table):
    mesh = plsc.VectorSubcoreMesh(core_axis_name="c", subcore_axis_name="s")
    f = functools.partial(
        pl.kernel, mesh=mesh,
        compiler_params=pltpu.CompilerParams(
            needs_layout_passes=False, use_tc_tiling_on_sc=True),
        out_type=jax.ShapeDtypeStruct((B,), jnp.float32),
        scratch_types=[
            pltpu.VMEM_SHARED((VOCAB,), jnp.float32),
            pltpu.VMEM((VOCAB,), jnp.float32),
            pltpu.VMEM((_CHUNK_ROWS, L), jnp.int32),
            pltpu.VMEM((_CHUNK_ROWS, L), jnp.int32),
            pltpu.VMEM(((_LANES - 1) * _TSTRIDE + _LANES,), jnp.float32),
            pltpu.VMEM((_ROWS_PER_W,), jnp.float32),
            pltpu.SemaphoreType.DMA,
            pltpu.SemaphoreType.DMA,
            pltpu.SemaphoreType.DMA,
        ],
    )(_sc_body)
    return f(predicted_ids, table)


# speedup vs baseline: 1.0422x; 1.0422x over previous
import functools

import jax
import jax.numpy as jnp
from jax import lax
from jax.experimental import pallas as pl
from jax.experimental.pallas import tpu as pltpu
from jax.experimental.pallas import tpu_sc as plsc

VOCAB = 100000
B = 4096
L = 200

_INFO = plsc.get_sparse_core_info()
_NC = _INFO.num_cores
_NS = _INFO.num_subcores
_NW = _NC * _NS
_LANES = _INFO.num_lanes

_ROWS_PER_W = B // _NW
_CHUNK_ROWS = 32
_NCHUNK = _ROWS_PER_W // _CHUNK_ROWS
_GPC = _CHUNK_ROWS // _LANES
_FULL = L // _LANES
_TAIL = L - _FULL * _LANES
_TSTRIDE = _LANES + 1


def _sc_body(ids_hbm, table_hbm, out_hbm, table_sp, table_v, ids_a, ids_b,
             tbuf, out_v, sem_t, sem_a, sem_b):
    sid = lax.axis_index("s")
    wid = sid * _NC + lax.axis_index("c")

    @pl.when(sid == 0)
    def _():
        pltpu.sync_copy(table_hbm, table_sp)

    plsc.subcore_barrier()
    tcopy = pltpu.make_async_copy(table_sp, table_v, sem_t)
    tcopy.start()

    base_row = pl.multiple_of(wid * _ROWS_PER_W, 8)
    bufs = [ids_a, ids_b]
    sems = [sem_a, sem_b]

    def chunk_copy(c):
        return pltpu.make_async_copy(
            ids_hbm.at[pl.ds(base_row + c * _CHUNK_ROWS, _CHUNK_ROWS), :],
            bufs[c % 2], sems[c % 2])

    chunk_copy(0).start()
    chunk_copy(1).start()

    lane = lax.broadcasted_iota(jnp.int32, (_LANES,), 0)
    tail_mask = lane >= (_LANES - _TAIL)
    zero = jnp.zeros((_LANES,), jnp.float32)

    for c in range(_NCHUNK):
        chunk_copy(c).wait()
        if c == 0:
            tcopy.wait()
        buf = bufs[c % 2]

        def row_body(a, carry, buf=buf):
            for g in range(_GPC):
                r = g * _LANES + a
                acc = zero
                for cc in range(_FULL):
                    ids16 = buf[r, pl.ds(cc * _LANES, _LANES)]
                    acc = acc + plsc.load_gather(table_v, [ids16])
                ids_t = buf[r, pl.ds(L - _LANES, _LANES)]
                vals_t = plsc.load_gather(table_v, [ids_t])
                acc = acc + jnp.where(tail_mask, vals_t, zero)
                tbuf[pl.ds((g * _LANES + a) * _TSTRIDE, _LANES)] = acc
            return carry

        lax.fori_loop(0, _LANES, row_body, 0, unroll=2)

        for g in range(_GPC):
            tot = zero
            for k in range(_LANES):
                tot = tot + plsc.load_gather(
                    tbuf, [(lane + g * _LANES) * _TSTRIDE + k])
            out_v[pl.ds(c * _CHUNK_ROWS + g * _LANES, _LANES)] = tot

        if c + 2 < _NCHUNK:
            chunk_copy(c + 2).start()

    base_out = pl.multiple_of(wid * _ROWS_PER_W, 8)
    pltpu.sync_copy(out_v, out_hbm.at[pl.ds(base_out, _ROWS_PER_W)])


@jax.jit
def kernel(predicted_ids, table):
    mesh = plsc.VectorSubcoreMesh(core_axis_name="c", subcore_axis_name="s")
    f = functools.partial(
        pl.kernel, mesh=mesh,
        compiler_params=pltpu.CompilerParams(needs_layout_passes=False),
        out_type=jax.ShapeDtypeStruct((B,), jnp.float32),
        scratch_types=[
            pltpu.VMEM_SHARED((VOCAB,), jnp.float32),
            pltpu.VMEM((VOCAB,), jnp.float32),
            pltpu.VMEM((_CHUNK_ROWS, L), jnp.int32),
            pltpu.VMEM((_CHUNK_ROWS, L), jnp.int32),
            pltpu.VMEM(((_CHUNK_ROWS - 1) * _TSTRIDE + _LANES,), jnp.float32),
            pltpu.VMEM((_ROWS_PER_W,), jnp.float32),
            pltpu.SemaphoreType.DMA,
            pltpu.SemaphoreType.DMA,
            pltpu.SemaphoreType.DMA,
        ],
    )(_sc_body)
    return f(predicted_ids, table)
